# Initial kernel scaffold; baseline (speedup 1.0000x reference)
import jax
import jax.numpy as jnp
from jax.experimental import pallas as pl


def kernel(support_xyz, support_features, batch_num_xyzs, new_xyz, new_xyz_grid_centers, batch_num_new_xyzs, W1, gamma1, beta1):
    raise NotImplementedError("write your pallas kernel here")


# speedup vs baseline: 20.3730x; 20.3730x over previous
import jax
import jax.numpy as jnp
from jax.experimental import pallas as pl

N_SUPPORT = 8192
M_NEW = 1024
G = 27
C = 128
R_NEIGH = 1.2 * 2.0
BN_EPS = 1e-5
CH = 128
MG = M_NEW * G
NCH = MG // CH


def _knn_mlp_kernel(cx_ref, cy_ref, cz_ref, px_ref, py_ref, pz_ref, cb_ref,
                    sx_ref, sy_ref, sz_ref, sb_ref, f_ref, wa_ref, wb_ref,
                    x_ref, sum_ref, ssq_ref):
    cxc = cx_ref[...].reshape(CH, 1)
    cyc = cy_ref[...].reshape(CH, 1)
    czc = cz_ref[...].reshape(CH, 1)
    pxc = px_ref[...].reshape(CH, 1)
    pyc = py_ref[...].reshape(CH, 1)
    pzc = pz_ref[...].reshape(CH, 1)
    cbc = cb_ref[...].reshape(CH, 1)
    sx = sx_ref[...]
    sy = sy_ref[...]
    sz = sz_ref[...]
    sb = sb_ref[...]

    cand = ((jnp.abs(pxc - sx) <= R_NEIGH)
            & (jnp.abs(pyc - sy) <= R_NEIGH)
            & (jnp.abs(pzc - sz) <= R_NEIGH)
            & (cbc == sb))
    gx = cxc - sx
    gy = cyc - sy
    gz = czc - sz
    d2 = gx * gx + gy * gy + gz * gz
    inf = jnp.float32(jnp.inf)
    d2m = jnp.where(cand, d2, inf)

    lane = jax.lax.broadcasted_iota(jnp.int32, (CH, N_SUPPORT), 1)
    big_i = jnp.int32(N_SUPPORT)

    vs = []
    ohs = []
    taken = jnp.zeros((CH, N_SUPPORT), dtype=jnp.bool_)
    for _ in range(3):
        masked = jnp.where(taken, inf, d2m)
        vmin = jnp.min(masked, axis=1, keepdims=True)
        is_min = (masked == vmin) & jnp.logical_not(taken)
        idx = jnp.min(jnp.where(is_min, lane, big_i), axis=1, keepdims=True)
        oh = lane == idx
        taken = taken | oh
        vs.append(vmin)
        ohs.append(oh)

    empty = vs[0] == inf
    zero = jnp.float32(0.0)
    recips = []
    for k in range(3):
        dk = jnp.where(vs[k] == inf, jnp.float32(1e10), vs[k])
        recips.append(1.0 / (dk + 1e-8))
    norm = recips[0] + recips[1] + recips[2]
    inv_norm = 1.0 / jnp.maximum(norm, 1e-8)

    s_mat = jnp.zeros((CH, N_SUPPORT), dtype=jnp.float32)
    for k in range(3):
        wk = recips[k] * inv_norm
        s_mat = s_mat + jnp.where(ohs[k], wk, zero)

    interp = jax.lax.dot_general(
        s_mat, f_ref[...], (((1,), (0,)), ((), ())),
        preferred_element_type=jnp.float32)

    x = jax.lax.dot_general(
        interp, wa_ref[...], (((1,), (0,)), ((), ())),
        preferred_element_type=jnp.float32)

    for k in range(3):
        ohk = ohs[k]
        nx = jnp.sum(jnp.where(ohk, sx, zero), axis=1, keepdims=True)
        ny = jnp.sum(jnp.where(ohk, sy, zero), axis=1, keepdims=True)
        nz = jnp.sum(jnp.where(ohk, sz, zero), axis=1, keepdims=True)
        x = x + (cxc - nx) * wb_ref[3 * k, :][None, :]
        x = x + (cyc - ny) * wb_ref[3 * k + 1, :][None, :]
        x = x + (czc - nz) * wb_ref[3 * k + 2, :][None, :]

    x = jnp.where(empty, zero, x)
    x_ref[...] = x
    sum_ref[...] = jnp.sum(x, axis=0, keepdims=True)[None]
    ssq_ref[...] = jnp.sum(x * x, axis=0, keepdims=True)[None]


def _bn_kernel(x_ref, sum_ref, ssq_ref, g_ref, b_ref, o_ref):
    n = jnp.float32(MG)
    mean = jnp.sum(sum_ref[...].reshape(NCH, C), axis=0, keepdims=True) / n
    ex2 = jnp.sum(ssq_ref[...].reshape(NCH, C), axis=0, keepdims=True) / n
    var = ex2 - mean * mean
    inv = jax.lax.rsqrt(var + BN_EPS)
    scale = inv * g_ref[...]
    shift = b_ref[...] - mean * scale
    o_ref[...] = jnp.maximum(x_ref[...] * scale + shift, 0.0)


def kernel(support_xyz, support_features, batch_num_xyzs, new_xyz,
           new_xyz_grid_centers, batch_num_new_xyzs, W1, gamma1, beta1):
    n = support_xyz.shape[0]
    m, g, _ = new_xyz_grid_centers.shape

    sup_batch = jnp.searchsorted(jnp.cumsum(batch_num_xyzs),
                                 jnp.arange(n, dtype=jnp.int32),
                                 side='right').astype(jnp.float32)
    new_batch = jnp.searchsorted(jnp.cumsum(batch_num_new_xyzs),
                                 jnp.arange(m, dtype=jnp.int32),
                                 side='right').astype(jnp.float32)

    centers = new_xyz_grid_centers.reshape(MG, 3)
    parents = jnp.repeat(new_xyz, g, axis=0)
    cb = jnp.repeat(new_batch, g).reshape(NCH, 1, CH)

    cx = centers[:, 0].reshape(NCH, 1, CH)
    cy = centers[:, 1].reshape(NCH, 1, CH)
    cz = centers[:, 2].reshape(NCH, 1, CH)
    px = parents[:, 0].reshape(NCH, 1, CH)
    py = parents[:, 1].reshape(NCH, 1, CH)
    pz = parents[:, 2].reshape(NCH, 1, CH)

    sx = support_xyz[:, 0].reshape(1, n)
    sy = support_xyz[:, 1].reshape(1, n)
    sz = support_xyz[:, 2].reshape(1, n)
    sb = sup_batch.reshape(1, n)

    wa = W1[:, :C].T
    wb = jnp.zeros((16, C), jnp.float32).at[:9, :].set(W1[:, C:C + 9].T)

    chunk_spec = pl.BlockSpec((1, 1, CH), lambda i: (i, 0, 0))
    full_row = pl.BlockSpec((1, n), lambda i: (0, 0))

    x, sums, ssqs = pl.pallas_call(
        _knn_mlp_kernel,
        grid=(NCH,),
        in_specs=[chunk_spec] * 7 + [full_row] * 4 + [
            pl.BlockSpec((n, C), lambda i: (0, 0)),
            pl.BlockSpec((C, C), lambda i: (0, 0)),
            pl.BlockSpec((16, C), lambda i: (0, 0)),
        ],
        out_specs=[
            pl.BlockSpec((CH, C), lambda i: (i, 0)),
            pl.BlockSpec((1, 1, C), lambda i: (i, 0, 0)),
            pl.BlockSpec((1, 1, C), lambda i: (i, 0, 0)),
        ],
        out_shape=[
            jax.ShapeDtypeStruct((MG, C), jnp.float32),
            jax.ShapeDtypeStruct((NCH, 1, C), jnp.float32),
            jax.ShapeDtypeStruct((NCH, 1, C), jnp.float32),
        ],
    )(cx, cy, cz, px, py, pz, cb, sx, sy, sz, sb,
      support_features, wa, wb)

    out = pl.pallas_call(
        _bn_kernel,
        grid=(NCH,),
        in_specs=[
            pl.BlockSpec((CH, C), lambda i: (i, 0)),
            pl.BlockSpec((NCH, 1, C), lambda i: (0, 0, 0)),
            pl.BlockSpec((NCH, 1, C), lambda i: (0, 0, 0)),
            pl.BlockSpec((1, C), lambda i: (0, 0)),
            pl.BlockSpec((1, C), lambda i: (0, 0)),
        ],
        out_specs=pl.BlockSpec((CH, C), lambda i: (i, 0)),
        out_shape=jax.ShapeDtypeStruct((MG, C), jnp.float32),
    )(x, sums, ssqs, gamma1.reshape(1, C), beta1.reshape(1, C))

    return out


# speedup vs baseline: 38.8706x; 1.9079x over previous
import jax
import jax.numpy as jnp
from jax.experimental import pallas as pl

N_SUPPORT = 8192
M_NEW = 1024
G = 27
C = 128
R_NEIGH = 1.2 * 2.0
BN_EPS = 1e-5
CH = 128
MG = M_NEW * G
NCH = MG // CH
NB = 2
NH = N_SUPPORT // NB
CPB = NCH // NB
EXT = 128
W_SCAN = NH + EXT
KBASE = float(2 ** 33)
KSTEP = 2048.0


def _knn_mlp_kernel(cx_ref, cy_ref, cz_ref, px_ref, py_ref, pz_ref,
                    sx_ref, sy_ref, sz_ref, gk_ref, f_ref, wa_ref, wb_ref,
                    x_ref, sum_ref, ssq_ref):
    cxc = cx_ref[...].reshape(CH, 1)
    cyc = cy_ref[...].reshape(CH, 1)
    czc = cz_ref[...].reshape(CH, 1)
    pxc = px_ref[...].reshape(CH, 1)
    pyc = py_ref[...].reshape(CH, 1)
    pzc = pz_ref[...].reshape(CH, 1)
    sx = sx_ref[...].reshape(1, W_SCAN)
    sy = sy_ref[...].reshape(1, W_SCAN)
    sz = sz_ref[...].reshape(1, W_SCAN)
    gkey = gk_ref[...].reshape(1, W_SCAN)

    lane = jax.lax.broadcasted_iota(jnp.int32, (CH, W_SCAN), 1)
    big_i = jnp.int32(W_SCAN)

    cand = ((jnp.abs(pxc - sx) <= R_NEIGH)
            & (jnp.abs(pyc - sy) <= R_NEIGH)
            & (jnp.abs(pzc - sz) <= R_NEIGH)
            & (lane < NH))
    gx = cxc - sx
    gy = cyc - sy
    gz = czc - sz
    d2 = gx * gx + gy * gy + gz * gz
    d2m = jnp.where(cand, d2, gkey)

    inf = jnp.float32(jnp.inf)
    kbase = jnp.float32(KBASE)
    vs = []
    ohs = []
    taken = jnp.zeros((CH, W_SCAN), dtype=jnp.bool_)
    for _ in range(3):
        masked = jnp.where(taken, inf, d2m)
        vmin = jnp.min(masked, axis=1, keepdims=True)
        is_min = (masked == vmin) & jnp.logical_not(taken)
        idx = jnp.min(jnp.where(is_min, lane, big_i), axis=1, keepdims=True)
        oh = lane == idx
        taken = taken | oh
        vs.append(vmin)
        ohs.append(oh)

    empty = vs[0] >= kbase
    zero = jnp.float32(0.0)
    recips = []
    for k in range(3):
        dk = jnp.where(vs[k] >= kbase, jnp.float32(1e10), vs[k])
        recips.append(1.0 / (dk + 1e-8))
    norm = recips[0] + recips[1] + recips[2]
    inv_norm = 1.0 / jnp.maximum(norm, 1e-8)

    s_mat = jnp.zeros((CH, W_SCAN), dtype=jnp.float32)
    for k in range(3):
        wk = recips[k] * inv_norm
        s_mat = s_mat + jnp.where(ohs[k], wk, zero)

    interp = jax.lax.dot_general(
        s_mat[:, :NH], f_ref[...], (((1,), (0,)), ((), ())),
        preferred_element_type=jnp.float32)

    x = jax.lax.dot_general(
        interp, wa_ref[...], (((1,), (0,)), ((), ())),
        preferred_element_type=jnp.float32)

    for k in range(3):
        ohk = ohs[k]
        nx = jnp.sum(jnp.where(ohk, sx, zero), axis=1, keepdims=True)
        ny = jnp.sum(jnp.where(ohk, sy, zero), axis=1, keepdims=True)
        nz = jnp.sum(jnp.where(ohk, sz, zero), axis=1, keepdims=True)
        x = x + (cxc - nx) * wb_ref[3 * k, :][None, :]
        x = x + (cyc - ny) * wb_ref[3 * k + 1, :][None, :]
        x = x + (czc - nz) * wb_ref[3 * k + 2, :][None, :]

    x = jnp.where(empty, zero, x)
    x_ref[...] = x
    sum_ref[...] = jnp.sum(x, axis=0, keepdims=True)[None]
    ssq_ref[...] = jnp.sum(x * x, axis=0, keepdims=True)[None]


def _bn_kernel(x_ref, sum_ref, ssq_ref, g_ref, b_ref, o_ref):
    n = jnp.float32(MG)
    mean = jnp.sum(sum_ref[...].reshape(NCH, C), axis=0, keepdims=True) / n
    ex2 = jnp.sum(ssq_ref[...].reshape(NCH, C), axis=0, keepdims=True) / n
    var = ex2 - mean * mean
    inv = jax.lax.rsqrt(var + BN_EPS)
    scale = inv * g_ref[...]
    shift = b_ref[...] - mean * scale
    o_ref[...] = jnp.maximum(x_ref[...] * scale + shift, 0.0)


def kernel(support_xyz, support_features, batch_num_xyzs, new_xyz,
           new_xyz_grid_centers, batch_num_new_xyzs, W1, gamma1, beta1):
    n = support_xyz.shape[0]
    m, g, _ = new_xyz_grid_centers.shape

    centers = new_xyz_grid_centers.reshape(MG, 3)
    parents = jnp.repeat(new_xyz, g, axis=0)

    cx = centers[:, 0].reshape(NCH, 1, CH)
    cy = centers[:, 1].reshape(NCH, 1, CH)
    cz = centers[:, 2].reshape(NCH, 1, CH)
    px = parents[:, 0].reshape(NCH, 1, CH)
    py = parents[:, 1].reshape(NCH, 1, CH)
    pz = parents[:, 2].reshape(NCH, 1, CH)

    def ext_rows(col):
        h0, h1 = col[:NH], col[NH:]
        return jnp.stack([jnp.concatenate([h0, h1[:EXT]]),
                          jnp.concatenate([h1, h0[:EXT]])]).reshape(NB, 1, W_SCAN)

    sxe = ext_rows(support_xyz[:, 0])
    sye = ext_rows(support_xyz[:, 1])
    sze = ext_rows(support_xyz[:, 2])
    loc = jnp.arange(NH, dtype=jnp.float32)
    ext = jnp.arange(EXT, dtype=jnp.float32)
    gkey = jnp.stack([
        jnp.concatenate([KBASE + loc * KSTEP, KBASE + (NH + ext) * KSTEP]),
        jnp.concatenate([KBASE + (NH + loc) * KSTEP, KBASE + ext * KSTEP]),
    ]).reshape(NB, 1, W_SCAN)

    wa = W1[:, :C].T
    wb = jnp.zeros((16, C), jnp.float32).at[:9, :].set(W1[:, C:C + 9].T)

    chunk_spec = pl.BlockSpec((1, 1, CH), lambda i: (i, 0, 0))
    batch_row = pl.BlockSpec((1, 1, W_SCAN), lambda i: (i // CPB, 0, 0))

    x, sums, ssqs = pl.pallas_call(
        _knn_mlp_kernel,
        grid=(NCH,),
        in_specs=[chunk_spec] * 6 + [batch_row] * 4 + [
            pl.BlockSpec((NH, C), lambda i: (i // CPB, 0)),
            pl.BlockSpec((C, C), lambda i: (0, 0)),
            pl.BlockSpec((16, C), lambda i: (0, 0)),
        ],
        out_specs=[
            pl.BlockSpec((CH, C), lambda i: (i, 0)),
            pl.BlockSpec((1, 1, C), lambda i: (i, 0, 0)),
            pl.BlockSpec((1, 1, C), lambda i: (i, 0, 0)),
        ],
        out_shape=[
            jax.ShapeDtypeStruct((MG, C), jnp.float32),
            jax.ShapeDtypeStruct((NCH, 1, C), jnp.float32),
            jax.ShapeDtypeStruct((NCH, 1, C), jnp.float32),
        ],
    )(cx, cy, cz, px, py, pz, sxe, sye, sze, gkey,
      support_features, wa, wb)

    out = pl.pallas_call(
        _bn_kernel,
        grid=(NCH,),
        in_specs=[
            pl.BlockSpec((CH, C), lambda i: (i, 0)),
            pl.BlockSpec((NCH, 1, C), lambda i: (0, 0, 0)),
            pl.BlockSpec((NCH, 1, C), lambda i: (0, 0, 0)),
            pl.BlockSpec((1, C), lambda i: (0, 0)),
            pl.BlockSpec((1, C), lambda i: (0, 0)),
        ],
        out_specs=pl.BlockSpec((CH, C), lambda i: (i, 0)),
        out_shape=jax.ShapeDtypeStruct((MG, C), jnp.float32),
    )(x, sums, ssqs, gamma1.reshape(1, C), beta1.reshape(1, C))

    return out
